# baseline (speedup 1.0000x reference)
import functools

import jax
import jax.numpy as jnp
from jax import lax
from jax.experimental import pallas as pl
from jax.experimental.pallas import tpu as pltpu
from jax.experimental.pallas import tpu_sc as plsc

NC = 2
NS = 16
NW = NC * NS
LANES = 16
K_BLK = 128


def _tile_slices(n):
    ch = (((n + NS - 1) // NS) + 7) // 8 * 8
    last = n - (NS - 1) * ch
    assert 0 < last <= ch and ch % 8 == 0 and last % 8 == 0
    return ch, last


def _lane_bcast(vec, lane):
    idx = jnp.full((LANES, 1), lane, jnp.int32)
    dnums = lax.GatherDimensionNumbers(
        offset_dims=(), collapsed_slice_dims=(0,), start_index_map=(0,))
    return lax.gather(vec, idx, dnums, (1,),
                      mode=lax.GatherScatterMode.PROMISE_IN_BOUNDS)


def _pack_bf16_pairs(g):
    r, d = g.shape
    a2 = g.astype(jnp.bfloat16).reshape(r, d // 32, 2, LANES)
    lo = lax.bitcast_convert_type(a2[:, :, 0, :], jnp.uint16).astype(jnp.int32)
    hi = lax.bitcast_convert_type(a2[:, :, 1, :], jnp.uint16).astype(jnp.int32)
    return lax.bitwise_or(lax.shift_left(hi, 16), lo).reshape(r, d // 2)


def _as_bf16(x):
    return lax.bitcast_convert_type(x, jnp.bfloat16).reshape(
        *x.shape[:-1], x.shape[-1] * 2)


def _scale_store(rows_ref, sc_ref, b, k, d, wk):
    for q in range(d // 32):
        x = rows_ref[b, k, pl.ds(q * 32, 32)]
        flo, fhi = plsc.unpack(x, format=plsc.PackFormat.INTERLEAVED)
        sc_ref[b, k, pl.ds(q * 32, LANES)] = flo * wk
        sc_ref[b, k, pl.ds(q * 32 + LANES, LANES)] = fhi * wk


def _zero_vmem_2d(ref, rows, d):
    zero16 = jnp.zeros((LANES,), jnp.float32)

    def body(r, carry):
        for q in range(d // LANES):
            ref[r, pl.ds(q * LANES, LANES)] = zero16
        return carry

    lax.fori_loop(0, rows, body, 0)


def _zero_vmem_1d(ref, total):
    zero16 = jnp.zeros((LANES,), jnp.float32)

    def body(i, carry):
        ref[pl.ds(i * LANES, LANES)] = zero16
        return carry

    lax.fori_loop(0, total // LANES, body, 0)


def _make_deg_kernel(n, nblk):
    ch, last = _tile_slices(n)
    chz = (ch + LANES - 1) // LANES * LANES
    mesh = plsc.VectorSubcoreMesh(core_axis_name="c", subcore_axis_name="s")

    @functools.partial(
        pl.kernel,
        out_type=jax.ShapeDtypeStruct((NC * n,), jnp.float32),
        mesh=mesh,
        scratch_types=[
            pltpu.VMEM((nblk, K_BLK), jnp.int32),
            pltpu.VMEM((nblk, K_BLK), jnp.float32),
            pltpu.VMEM((chz,), jnp.float32),
            pltpu.VMEM_SHARED((n,), jnp.float32),
            pltpu.SemaphoreType.DMA,
        ],
        compiler_params=pltpu.CompilerParams(use_tc_tiling_on_sc=False,
                                             needs_layout_passes=False),
    )
    def deg_kernel(col_hbm, w_hbm, out_hbm, col_v, w_v, zed_v, acc_sh, sem):
        c = lax.axis_index("c")
        s = lax.axis_index("s")
        wid = c * NS + s

        _zero_vmem_1d(zed_v, chz)

        @pl.when(s < NS - 1)
        def _():
            pltpu.sync_copy(zed_v.at[pl.ds(0, ch)], acc_sh.at[pl.ds(s * ch, ch)])

        @pl.when(s == NS - 1)
        def _():
            pltpu.sync_copy(zed_v.at[pl.ds(0, last)],
                            acc_sh.at[pl.ds((NS - 1) * ch, last)])

        pltpu.sync_copy(col_hbm.at[wid], col_v)
        pltpu.sync_copy(w_hbm.at[wid], w_v)
        plsc.subcore_barrier()

        GRP = 8

        def grp(gg, carry):
            for b in range(GRP):
                pltpu.async_copy(w_v.at[gg * GRP + b],
                                 acc_sh.at[col_v.at[gg * GRP + b]], sem,
                                 add=True)
            for b in range(GRP):
                pltpu.make_async_copy(w_v.at[gg * GRP + b],
                                      acc_sh.at[col_v.at[gg * GRP + b]],
                                      sem).wait()
            return carry

        assert nblk % GRP == 0
        lax.fori_loop(0, nblk // GRP, grp, 0)
        plsc.subcore_barrier()

        @pl.when(s < NS - 1)
        def _():
            pltpu.sync_copy(acc_sh.at[pl.ds(s * ch, ch)], zed_v.at[pl.ds(0, ch)])
            pltpu.sync_copy(zed_v.at[pl.ds(0, ch)],
                            out_hbm.at[pl.ds(c * n + s * ch, ch)])

        @pl.when(s == NS - 1)
        def _():
            pltpu.sync_copy(acc_sh.at[pl.ds((NS - 1) * ch, last)],
                            zed_v.at[pl.ds(0, last)])
            pltpu.sync_copy(zed_v.at[pl.ds(0, last)],
                            out_hbm.at[pl.ds(c * n + (NS - 1) * ch, last)])

    return deg_kernel


def _chunk_list(total, zr):
    k, rem = divmod(total, zr)
    return [(i * zr, zr) for i in range(k)] + ([(k * zr, rem)] if rem else [])


ZR = 128


def _make_edge_kernel(n, nblk, d):
    assert d % LANES == 0 and nblk % 2 == 0
    ch, last = _tile_slices(n)
    mesh = plsc.VectorSubcoreMesh(core_axis_name="c", subcore_axis_name="s")

    @functools.partial(
        pl.kernel,
        out_type=jax.ShapeDtypeStruct((n, NC * d), jnp.float32),
        mesh=mesh,
        scratch_types=[
            pltpu.VMEM((nblk, K_BLK), jnp.int32),
            pltpu.VMEM((nblk, K_BLK), jnp.int32),
            pltpu.VMEM((nblk, K_BLK), jnp.float32),
            pltpu.VMEM((2, K_BLK, d), jnp.bfloat16),
            pltpu.VMEM((2, K_BLK, d), jnp.float32),
            pltpu.VMEM((ZR, d), jnp.float32),
            pltpu.VMEM_SHARED((n, d), jnp.float32),
            pltpu.SemaphoreType.DMA,
            pltpu.SemaphoreType.DMA,
            pltpu.SemaphoreType.DMA,
            pltpu.SemaphoreType.DMA,
        ],
        compiler_params=pltpu.CompilerParams(use_tc_tiling_on_sc=False,
                                             needs_layout_passes=False),
    )
    def edge_kernel(g_hbm, row_hbm, col_hbm, w_hbm, out_hbm,
                    row_v, col_v, w_v, rows_v, sc_v, zed_v, acc_sh,
                    gsem0, gsem1, ssem0, ssem1):
        gsem = (gsem0, gsem1)
        ssem = (ssem0, ssem1)
        c = lax.axis_index("c")
        s = lax.axis_index("s")
        wid = c * NS + s

        _zero_vmem_2d(zed_v, ZR, d)

        @pl.when(s < NS - 1)
        def _():
            for off, sz in _chunk_list(ch, ZR):
                pltpu.sync_copy(zed_v.at[pl.ds(0, sz)],
                                acc_sh.at[pl.ds(s * ch + off, sz)])

        @pl.when(s == NS - 1)
        def _():
            for off, sz in _chunk_list(last, ZR):
                pltpu.sync_copy(zed_v.at[pl.ds(0, sz)],
                                acc_sh.at[pl.ds((NS - 1) * ch + off, sz)])

        pltpu.sync_copy(row_hbm.at[wid], row_v)
        pltpu.sync_copy(col_hbm.at[wid], col_v)
        pltpu.sync_copy(w_hbm.at[wid], w_v)
        plsc.subcore_barrier()

        for b in range(2):
            pltpu.async_copy(g_hbm.at[row_v.at[b]], rows_v.at[b], gsem[b])

        def blk2(j0, carry):
            for b in range(2):
                j = j0 * 2 + b
                pltpu.make_async_copy(g_hbm.at[row_v.at[j]], rows_v.at[b],
                                      gsem[b]).wait()

                @pl.when(j0 > 0)
                def _():
                    jp = j - 2
                    pltpu.make_async_copy(sc_v.at[b],
                                          acc_sh.at[col_v.at[jp]],
                                          ssem[b]).wait()

                def scale(kb, carry2):
                    w16 = w_v[j, pl.ds(kb * LANES, LANES)]
                    for jj in range(LANES):
                        wk = _lane_bcast(w16, jj)
                        _scale_store(rows_v, sc_v, b, kb * LANES + jj, d, wk)
                    return carry2

                lax.fori_loop(0, K_BLK // LANES, scale, 0)
                pltpu.async_copy(sc_v.at[b], acc_sh.at[col_v.at[j]],
                                 ssem[b], add=True)

                @pl.when(j + 2 < nblk)
                def _():
                    pltpu.async_copy(g_hbm.at[row_v.at[j + 2]], rows_v.at[b],
                                     gsem[b])
            return carry

        lax.fori_loop(0, nblk // 2, blk2, 0)
        for b in range(2):
            pltpu.make_async_copy(sc_v.at[b],
                                  acc_sh.at[col_v.at[nblk - 2 + b]],
                                  ssem[b]).wait()
        plsc.subcore_barrier()

        osl = pl.ds(c * d, d)

        @pl.when(s < NS - 1)
        def _():
            for off, sz in _chunk_list(ch, ZR):
                pltpu.sync_copy(acc_sh.at[pl.ds(s * ch + off, sz)],
                                zed_v.at[pl.ds(0, sz)])
                pltpu.sync_copy(zed_v.at[pl.ds(0, sz)],
                                out_hbm.at[pl.ds(s * ch + off, sz), osl])

        @pl.when(s == NS - 1)
        def _():
            for off, sz in _chunk_list(last, ZR):
                pltpu.sync_copy(acc_sh.at[pl.ds((NS - 1) * ch + off, sz)],
                                zed_v.at[pl.ds(0, sz)])
                pltpu.sync_copy(zed_v.at[pl.ds(0, sz)],
                                out_hbm.at[pl.ds((NS - 1) * ch + off, sz), osl])

    return edge_kernel


GI = 8


def _make_edge1_kernel(n, nblk2, d):
    assert d % LANES == 0 and nblk2 % (2 * GI) == 0 and nblk2 // GI >= 2
    ch, last = _tile_slices(n)
    ngrp = nblk2 // GI
    mesh = plsc.VectorSubcoreMesh(core_axis_name="c", subcore_axis_name="s")

    @functools.partial(
        pl.kernel,
        out_type=jax.ShapeDtypeStruct((n, NC * d), jnp.float32),
        mesh=mesh,
        scratch_types=[
            pltpu.VMEM((2, GI, K_BLK), jnp.int32),
            pltpu.VMEM((2, GI, K_BLK), jnp.int32),
            pltpu.VMEM((2, GI, K_BLK), jnp.float32),
            pltpu.VMEM((2, K_BLK, d), jnp.bfloat16),
            pltpu.VMEM((2, K_BLK, d), jnp.float32),
            pltpu.VMEM((ZR, d), jnp.float32),
            pltpu.VMEM_SHARED((n, d), jnp.float32),
            pltpu.SemaphoreType.DMA,
            pltpu.SemaphoreType.DMA,
            pltpu.SemaphoreType.DMA,
            pltpu.SemaphoreType.DMA,
            pltpu.SemaphoreType.DMA,
            pltpu.SemaphoreType.DMA,
        ],
        compiler_params=pltpu.CompilerParams(use_tc_tiling_on_sc=False,
                                             needs_layout_passes=False),
    )
    def edge1_kernel(g_hbm, row_hbm, col_hbm, w_hbm, out_hbm,
                     row_v, col_v, w_v, rows_v, sc_v, zed_v, acc_sh,
                     gsem0, gsem1, ssem0, ssem1, isem0, isem1):
        gsem = (gsem0, gsem1)
        ssem = (ssem0, ssem1)
        isem = (isem0, isem1)
        c = lax.axis_index("c")
        s = lax.axis_index("s")
        gsl = pl.ds(c * d, d)
        cn16 = jnp.full((LANES,), c * n, jnp.int32)

        _zero_vmem_2d(zed_v, ZR, d)

        @pl.when(s < NS - 1)
        def _():
            for off, sz in _chunk_list(ch, ZR):
                pltpu.sync_copy(zed_v.at[pl.ds(0, sz)],
                                acc_sh.at[pl.ds(s * ch + off, sz)])

        @pl.when(s == NS - 1)
        def _():
            for off, sz in _chunk_list(last, ZR):
                pltpu.sync_copy(zed_v.at[pl.ds(0, sz)],
                                acc_sh.at[pl.ds((NS - 1) * ch + off, sz)])

        def offset_rows(buf):
            def obody(q, carry):
                r = q // (K_BLK // LANES)
                o = (q % (K_BLK // LANES)) * LANES
                sl = pl.ds(o, LANES)
                row_v[buf, r, sl] = row_v[buf, r, sl] + cn16
                return carry
            lax.fori_loop(0, GI * K_BLK // LANES, obody, 0)

        def stage_group(g, buf, sem):
            pltpu.async_copy(row_hbm.at[s, pl.ds(g * GI, GI)], row_v.at[buf], sem)
            pltpu.async_copy(col_hbm.at[s, pl.ds(g * GI, GI)], col_v.at[buf], sem)
            pltpu.async_copy(w_hbm.at[s, pl.ds(g * GI, GI)], w_v.at[buf], sem)

        def wait_group(g, buf, sem):
            pltpu.make_async_copy(row_hbm.at[s, pl.ds(g * GI, GI)],
                                  row_v.at[buf], sem).wait()
            pltpu.make_async_copy(col_hbm.at[s, pl.ds(g * GI, GI)],
                                  col_v.at[buf], sem).wait()
            pltpu.make_async_copy(w_hbm.at[s, pl.ds(g * GI, GI)],
                                  w_v.at[buf], sem).wait()

        stage_group(0, 0, isem[0])
        wait_group(0, 0, isem[0])
        offset_rows(0)
        plsc.subcore_barrier()

        for b in range(2):
            pltpu.async_copy(g_hbm.at[row_v.at[0, b]], rows_v.at[b], gsem[b])

        def grouppair(g0, carry):
            for bgi in range(2):
                g = g0 * 2 + bgi
                for jb in range(GI):
                    b = jb % 2
                    j = g * GI + jb
                    pltpu.make_async_copy(g_hbm.at[row_v.at[bgi, jb]],
                                          rows_v.at[b], gsem[b]).wait()
                    if jb >= 2:
                        cprev = col_v.at[bgi, jb - 2]
                    else:
                        cprev = col_v.at[1 - bgi, GI - 2 + jb]

                    @pl.when(j >= 2)
                    def _(cprev=cprev, b=b):
                        pltpu.make_async_copy(sc_v.at[b], acc_sh.at[cprev],
                                              ssem[b]).wait()

                    def scale(kb, carry2, bgi=bgi, jb=jb, b=b):
                        w16 = w_v[bgi, jb, pl.ds(kb * LANES, LANES)]
                        for jj in range(LANES):
                            wk = _lane_bcast(w16, jj)
                            _scale_store(rows_v, sc_v, b, kb * LANES + jj, d,
                                         wk)
                        return carry2

                    lax.fori_loop(0, K_BLK // LANES, scale, 0)
                    pltpu.async_copy(sc_v.at[b], acc_sh.at[col_v.at[bgi, jb]],
                                     ssem[b], add=True)

                    if jb == 2:
                        @pl.when(g + 1 < ngrp)
                        def _(g=g, bgi=bgi):
                            stage_group(g + 1, 1 - bgi, isem[1 - bgi])

                    if jb == GI - 2:
                        @pl.when(g + 1 < ngrp)
                        def _(g=g, bgi=bgi):
                            wait_group(g + 1, 1 - bgi, isem[1 - bgi])
                            offset_rows(1 - bgi)

                    if jb <= GI - 3:
                        nidx = row_v.at[bgi, jb + 2]
                    else:
                        nidx = row_v.at[1 - bgi, jb + 2 - GI]

                    @pl.when(j + 2 < nblk2)
                    def _(nidx=nidx, b=b):
                        pltpu.async_copy(g_hbm.at[nidx], rows_v.at[b], gsem[b])
            return carry

        lax.fori_loop(0, ngrp // 2, grouppair, 0)
        for b in range(2):
            pltpu.make_async_copy(sc_v.at[b],
                                  acc_sh.at[col_v.at[1, GI - 2 + b]],
                                  ssem[b]).wait()
        plsc.subcore_barrier()

        @pl.when(s < NS - 1)
        def _():
            for off, sz in _chunk_list(ch, ZR):
                pltpu.sync_copy(acc_sh.at[pl.ds(s * ch + off, sz)],
                                zed_v.at[pl.ds(0, sz)])
                pltpu.sync_copy(zed_v.at[pl.ds(0, sz)],
                                out_hbm.at[pl.ds(s * ch + off, sz), gsl])

        @pl.when(s == NS - 1)
        def _():
            for off, sz in _chunk_list(last, ZR):
                pltpu.sync_copy(acc_sh.at[pl.ds((NS - 1) * ch + off, sz)],
                                zed_v.at[pl.ds(0, sz)])
                pltpu.sync_copy(zed_v.at[pl.ds(0, sz)],
                                out_hbm.at[pl.ds((NS - 1) * ch + off, sz), gsl])

    return edge1_kernel


def _dis_from(deg_ref):
    deg = deg_ref[:, 0] + deg_ref[:, 1] + 1.0
    return jnp.where(deg > 0, lax.rsqrt(deg), 0.0)


def _make_prep_body(dch):
    def body(deg_ref, x_ref, w_ref, gf_ref, gcm_ref):
        dis = _dis_from(deg_ref)
        h = jnp.dot(x_ref[...], w_ref[...], preferred_element_type=jnp.float32)
        g = h * dis[:, None]
        gf_ref[...] = g
        for p in range(NC):
            gcm_ref[p] = _pack_bf16_pairs(g[:, p * dch:(p + 1) * dch])
    return body


def _mid_body(deg_ref, s_ref, g1_ref, b1_ref, w2_ref, g2_ref, g2b_ref):
    dis = _dis_from(deg_ref)
    t = (s_ref[...] + g1_ref[...]) * dis[:, None] + b1_ref[...]
    t = jnp.maximum(t, 0.0)
    h2 = jnp.dot(t, w2_ref[...], preferred_element_type=jnp.float32)
    g2 = h2 * dis[:, None]
    g2_ref[...] = g2
    g2b_ref[...] = _pack_bf16_pairs(g2)


def _make_fin_body(d2):
    def body(deg_ref, s_ref, g2_ref, b2_ref, o_ref):
        dis = _dis_from(deg_ref)
        s = s_ref[:, :d2] + s_ref[:, d2:NC * d2]
        o_ref[...] = (s + g2_ref[...]) * dis[:, None] + b2_ref[...]
    return body


def _row_blocks(n):
    for blk in (2000, 1000, 500, 250, 125, n):
        if n % blk == 0 and blk % 8 == 0:
            return blk, n // blk
    return n, 1


def kernel(x, edge_index, edge_weight, W1, b1, W2, b2):
    n, f = x.shape
    d1 = W1.shape[1]
    d2 = W2.shape[1]
    e = edge_weight.shape[0]

    row = edge_index[0].astype(jnp.int32)
    col = edge_index[1].astype(jnp.int32)
    w = edge_weight.astype(jnp.float32)

    emult = NS * K_BLK * GI * 2
    e_pad = (e + emult - 1) // emult * emult
    if e_pad != e:
        extra = e_pad - e
        pad_idx = (jnp.arange(extra, dtype=jnp.int32) * 16) % n
        row = jnp.concatenate([row, pad_idx])
        col = jnp.concatenate([col, pad_idx])
        w = jnp.concatenate([w, jnp.zeros((extra,), jnp.float32)])
    nblk = e_pad // (NW * K_BLK)
    nblk2 = e_pad // (NS * K_BLK)
    row3 = row.reshape(NW, nblk, K_BLK)
    col3 = col.reshape(NW, nblk, K_BLK)
    w3 = w.reshape(NW, nblk, K_BLK)
    row2 = row.reshape(NS, nblk2, K_BLK)
    col2 = col.reshape(NS, nblk2, K_BLK)
    w2 = w.reshape(NS, nblk2, K_BLK)

    deg_parts = _make_deg_kernel(n, nblk)(col3, w3)
    deg_nt = deg_parts.reshape(NC, n).T

    blk, nrblk = _row_blocks(n)
    full2 = lambda i: (0, 0)

    DCH = d1 // NC
    assert d1 == NC * DCH and d2 == DCH

    g1f, g1cm = pl.pallas_call(
        _make_prep_body(DCH),
        grid=(nrblk,),
        in_specs=[
            pl.BlockSpec((blk, NC), lambda i: (i, 0)),
            pl.BlockSpec((blk, f), lambda i: (i, 0)),
            pl.BlockSpec((f, d1), full2),
        ],
        out_specs=[
            pl.BlockSpec((blk, d1), lambda i: (i, 0)),
            pl.BlockSpec((NC, blk, DCH // 2), lambda i: (0, i, 0)),
        ],
        out_shape=[
            jax.ShapeDtypeStruct((n, d1), jnp.float32),
            jax.ShapeDtypeStruct((NC, n, DCH // 2), jnp.int32),
        ],
    )(deg_nt, x, W1)

    s1 = _make_edge1_kernel(n, nblk2, DCH)(
        _as_bf16(g1cm).reshape(NC * n, DCH), row2, col2, w2)

    g2, g2b = pl.pallas_call(
        _mid_body,
        grid=(nrblk,),
        in_specs=[
            pl.BlockSpec((blk, NC), lambda i: (i, 0)),
            pl.BlockSpec((blk, d1), lambda i: (i, 0)),
            pl.BlockSpec((blk, d1), lambda i: (i, 0)),
            pl.BlockSpec((1, d1), full2),
            pl.BlockSpec((d1, d2), full2),
        ],
        out_specs=[
            pl.BlockSpec((blk, d2), lambda i: (i, 0)),
            pl.BlockSpec((blk, d2 // 2), lambda i: (i, 0)),
        ],
        out_shape=[
            jax.ShapeDtypeStruct((n, d2), jnp.float32),
            jax.ShapeDtypeStruct((n, d2 // 2), jnp.int32),
        ],
    )(deg_nt, s1, g1f, b1.reshape(1, d1), W2)

    s2 = _make_edge_kernel(n, nblk, d2)(_as_bf16(g2b), row3, col3, w3)

    out = pl.pallas_call(
        _make_fin_body(d2),
        grid=(nrblk,),
        in_specs=[
            pl.BlockSpec((blk, NC), lambda i: (i, 0)),
            pl.BlockSpec((blk, NC * d2), lambda i: (i, 0)),
            pl.BlockSpec((blk, d2), lambda i: (i, 0)),
            pl.BlockSpec((1, d2), full2),
        ],
        out_specs=pl.BlockSpec((blk, d2), lambda i: (i, 0)),
        out_shape=jax.ShapeDtypeStruct((n, d2), jnp.float32),
    )(deg_nt, s2, g2, b2.reshape(1, d2))

    return out


# speedup vs baseline: 1.1210x; 1.1210x over previous
import functools

import jax
import jax.numpy as jnp
from jax import lax
from jax.experimental import pallas as pl
from jax.experimental.pallas import tpu as pltpu
from jax.experimental.pallas import tpu_sc as plsc

NC = 2
NS = 16
NW = NC * NS
LANES = 16
K_BLK = 128


def _tile_slices(n):
    ch = (((n + NS - 1) // NS) + 7) // 8 * 8
    last = n - (NS - 1) * ch
    assert 0 < last <= ch and ch % 8 == 0 and last % 8 == 0
    return ch, last


def _lane_bcast(vec, lane):
    idx = jnp.full((LANES, 1), lane, jnp.int32)
    dnums = lax.GatherDimensionNumbers(
        offset_dims=(), collapsed_slice_dims=(0,), start_index_map=(0,))
    return lax.gather(vec, idx, dnums, (1,),
                      mode=lax.GatherScatterMode.PROMISE_IN_BOUNDS)


def _pack_bf16_pairs(g):
    r, d = g.shape
    a2 = g.astype(jnp.bfloat16).reshape(r, d // 32, 2, LANES)
    lo = lax.bitcast_convert_type(a2[:, :, 0, :], jnp.uint16).astype(jnp.int32)
    hi = lax.bitcast_convert_type(a2[:, :, 1, :], jnp.uint16).astype(jnp.int32)
    return lax.bitwise_or(lax.shift_left(hi, 16), lo).reshape(r, d // 2)


def _as_bf16(x):
    return lax.bitcast_convert_type(x, jnp.bfloat16).reshape(
        *x.shape[:-1], x.shape[-1] * 2)


def _scale_store(rows_ref, sc_ref, b, k, d, wk):
    mask = jnp.full((LANES,), -65536, jnp.int32)
    for q in range(d // 32):
        wbits = rows_ref[b, k, pl.ds(q * LANES, LANES)]
        flo = plsc.bitcast(lax.shift_left(wbits, 16), jnp.float32)
        fhi = plsc.bitcast(lax.bitwise_and(wbits, mask), jnp.float32)
        sc_ref[b, k, pl.ds(q * 32, LANES)] = flo * wk
        sc_ref[b, k, pl.ds(q * 32 + LANES, LANES)] = fhi * wk


def _zero_vmem_2d(ref, rows, d):
    zero16 = jnp.zeros((LANES,), jnp.float32)

    def body(r, carry):
        for q in range(d // LANES):
            ref[r, pl.ds(q * LANES, LANES)] = zero16
        return carry

    lax.fori_loop(0, rows, body, 0)


def _zero_vmem_1d(ref, total):
    zero16 = jnp.zeros((LANES,), jnp.float32)

    def body(i, carry):
        ref[pl.ds(i * LANES, LANES)] = zero16
        return carry

    lax.fori_loop(0, total // LANES, body, 0)


def _make_deg_kernel(n, nblk):
    ch, last = _tile_slices(n)
    chz = (ch + LANES - 1) // LANES * LANES
    mesh = plsc.VectorSubcoreMesh(core_axis_name="c", subcore_axis_name="s")

    @functools.partial(
        pl.kernel,
        out_type=jax.ShapeDtypeStruct((NC * n,), jnp.float32),
        mesh=mesh,
        scratch_types=[
            pltpu.VMEM((nblk, K_BLK), jnp.int32),
            pltpu.VMEM((nblk, K_BLK), jnp.float32),
            pltpu.VMEM((chz,), jnp.float32),
            pltpu.VMEM_SHARED((n,), jnp.float32),
            pltpu.SemaphoreType.DMA,
        ],
        compiler_params=pltpu.CompilerParams(use_tc_tiling_on_sc=False,
                                             needs_layout_passes=False),
    )
    def deg_kernel(col_hbm, w_hbm, out_hbm, col_v, w_v, zed_v, acc_sh, sem):
        c = lax.axis_index("c")
        s = lax.axis_index("s")
        wid = c * NS + s

        _zero_vmem_1d(zed_v, chz)

        @pl.when(s < NS - 1)
        def _():
            pltpu.sync_copy(zed_v.at[pl.ds(0, ch)], acc_sh.at[pl.ds(s * ch, ch)])

        @pl.when(s == NS - 1)
        def _():
            pltpu.sync_copy(zed_v.at[pl.ds(0, last)],
                            acc_sh.at[pl.ds((NS - 1) * ch, last)])

        pltpu.sync_copy(col_hbm.at[wid], col_v)
        pltpu.sync_copy(w_hbm.at[wid], w_v)
        plsc.subcore_barrier()

        GRP = 8

        def grp(gg, carry):
            for b in range(GRP):
                pltpu.async_copy(w_v.at[gg * GRP + b],
                                 acc_sh.at[col_v.at[gg * GRP + b]], sem,
                                 add=True)
            for b in range(GRP):
                pltpu.make_async_copy(w_v.at[gg * GRP + b],
                                      acc_sh.at[col_v.at[gg * GRP + b]],
                                      sem).wait()
            return carry

        assert nblk % GRP == 0
        lax.fori_loop(0, nblk // GRP, grp, 0)
        plsc.subcore_barrier()

        @pl.when(s < NS - 1)
        def _():
            pltpu.sync_copy(acc_sh.at[pl.ds(s * ch, ch)], zed_v.at[pl.ds(0, ch)])
            pltpu.sync_copy(zed_v.at[pl.ds(0, ch)],
                            out_hbm.at[pl.ds(c * n + s * ch, ch)])

        @pl.when(s == NS - 1)
        def _():
            pltpu.sync_copy(acc_sh.at[pl.ds((NS - 1) * ch, last)],
                            zed_v.at[pl.ds(0, last)])
            pltpu.sync_copy(zed_v.at[pl.ds(0, last)],
                            out_hbm.at[pl.ds(c * n + (NS - 1) * ch, last)])

    return deg_kernel


def _chunk_list(total, zr):
    k, rem = divmod(total, zr)
    return [(i * zr, zr) for i in range(k)] + ([(k * zr, rem)] if rem else [])


ZR = 128


def _make_edge_kernel(n, nblk, d):
    assert d % LANES == 0 and nblk % 2 == 0
    ch, last = _tile_slices(n)
    mesh = plsc.VectorSubcoreMesh(core_axis_name="c", subcore_axis_name="s")

    @functools.partial(
        pl.kernel,
        out_type=jax.ShapeDtypeStruct((n, NC * d), jnp.float32),
        mesh=mesh,
        scratch_types=[
            pltpu.VMEM((nblk, K_BLK), jnp.int32),
            pltpu.VMEM((nblk, K_BLK), jnp.int32),
            pltpu.VMEM((nblk, K_BLK), jnp.float32),
            pltpu.VMEM((2, K_BLK, d // 2), jnp.int32),
            pltpu.VMEM((2, K_BLK, d), jnp.float32),
            pltpu.VMEM((ZR, d), jnp.float32),
            pltpu.VMEM_SHARED((n, d), jnp.float32),
            pltpu.SemaphoreType.DMA,
            pltpu.SemaphoreType.DMA,
            pltpu.SemaphoreType.DMA,
            pltpu.SemaphoreType.DMA,
        ],
        compiler_params=pltpu.CompilerParams(use_tc_tiling_on_sc=False,
                                             needs_layout_passes=False),
    )
    def edge_kernel(g_hbm, row_hbm, col_hbm, w_hbm, out_hbm,
                    row_v, col_v, w_v, rows_v, sc_v, zed_v, acc_sh,
                    gsem0, gsem1, ssem0, ssem1):
        gsem = (gsem0, gsem1)
        ssem = (ssem0, ssem1)
        c = lax.axis_index("c")
        s = lax.axis_index("s")
        wid = c * NS + s

        _zero_vmem_2d(zed_v, ZR, d)

        @pl.when(s < NS - 1)
        def _():
            for off, sz in _chunk_list(ch, ZR):
                pltpu.sync_copy(zed_v.at[pl.ds(0, sz)],
                                acc_sh.at[pl.ds(s * ch + off, sz)])

        @pl.when(s == NS - 1)
        def _():
            for off, sz in _chunk_list(last, ZR):
                pltpu.sync_copy(zed_v.at[pl.ds(0, sz)],
                                acc_sh.at[pl.ds((NS - 1) * ch + off, sz)])

        pltpu.sync_copy(row_hbm.at[wid], row_v)
        pltpu.sync_copy(col_hbm.at[wid], col_v)
        pltpu.sync_copy(w_hbm.at[wid], w_v)
        plsc.subcore_barrier()

        for b in range(2):
            pltpu.async_copy(g_hbm.at[row_v.at[b]], rows_v.at[b], gsem[b])

        def blk2(j0, carry):
            for b in range(2):
                j = j0 * 2 + b
                pltpu.make_async_copy(g_hbm.at[row_v.at[j]], rows_v.at[b],
                                      gsem[b]).wait()

                @pl.when(j0 > 0)
                def _():
                    jp = j - 2
                    pltpu.make_async_copy(sc_v.at[b],
                                          acc_sh.at[col_v.at[jp]],
                                          ssem[b]).wait()

                def scale(kb, carry2):
                    w16 = w_v[j, pl.ds(kb * LANES, LANES)]
                    for jj in range(LANES):
                        wk = _lane_bcast(w16, jj)
                        _scale_store(rows_v, sc_v, b, kb * LANES + jj, d, wk)
                    return carry2

                lax.fori_loop(0, K_BLK // LANES, scale, 0)
                pltpu.async_copy(sc_v.at[b], acc_sh.at[col_v.at[j]],
                                 ssem[b], add=True)

                @pl.when(j + 2 < nblk)
                def _():
                    pltpu.async_copy(g_hbm.at[row_v.at[j + 2]], rows_v.at[b],
                                     gsem[b])
            return carry

        lax.fori_loop(0, nblk // 2, blk2, 0)
        for b in range(2):
            pltpu.make_async_copy(sc_v.at[b],
                                  acc_sh.at[col_v.at[nblk - 2 + b]],
                                  ssem[b]).wait()
        plsc.subcore_barrier()

        osl = pl.ds(c * d, d)

        @pl.when(s < NS - 1)
        def _():
            for off, sz in _chunk_list(ch, ZR):
                pltpu.sync_copy(acc_sh.at[pl.ds(s * ch + off, sz)],
                                zed_v.at[pl.ds(0, sz)])
                pltpu.sync_copy(zed_v.at[pl.ds(0, sz)],
                                out_hbm.at[pl.ds(s * ch + off, sz), osl])

        @pl.when(s == NS - 1)
        def _():
            for off, sz in _chunk_list(last, ZR):
                pltpu.sync_copy(acc_sh.at[pl.ds((NS - 1) * ch + off, sz)],
                                zed_v.at[pl.ds(0, sz)])
                pltpu.sync_copy(zed_v.at[pl.ds(0, sz)],
                                out_hbm.at[pl.ds((NS - 1) * ch + off, sz), osl])

    return edge_kernel


GI = 8


def _make_edge1_kernel(n, nblk2, d):
    assert d % LANES == 0 and nblk2 % (2 * GI) == 0 and nblk2 // GI >= 2
    ch, last = _tile_slices(n)
    ngrp = nblk2 // GI
    mesh = plsc.VectorSubcoreMesh(core_axis_name="c", subcore_axis_name="s")

    @functools.partial(
        pl.kernel,
        out_type=jax.ShapeDtypeStruct((n, NC * d), jnp.float32),
        mesh=mesh,
        scratch_types=[
            pltpu.VMEM((2, GI, K_BLK), jnp.int32),
            pltpu.VMEM((2, GI, K_BLK), jnp.int32),
            pltpu.VMEM((2, GI, K_BLK), jnp.float32),
            pltpu.VMEM((2, K_BLK, d // 2), jnp.int32),
            pltpu.VMEM((2, K_BLK, d), jnp.float32),
            pltpu.VMEM((ZR, d), jnp.float32),
            pltpu.VMEM_SHARED((n, d), jnp.float32),
            pltpu.SemaphoreType.DMA,
            pltpu.SemaphoreType.DMA,
            pltpu.SemaphoreType.DMA,
            pltpu.SemaphoreType.DMA,
            pltpu.SemaphoreType.DMA,
            pltpu.SemaphoreType.DMA,
        ],
        compiler_params=pltpu.CompilerParams(use_tc_tiling_on_sc=False,
                                             needs_layout_passes=False),
    )
    def edge1_kernel(g_hbm, row_hbm, col_hbm, w_hbm, out_hbm,
                     row_v, col_v, w_v, rows_v, sc_v, zed_v, acc_sh,
                     gsem0, gsem1, ssem0, ssem1, isem0, isem1):
        gsem = (gsem0, gsem1)
        ssem = (ssem0, ssem1)
        isem = (isem0, isem1)
        c = lax.axis_index("c")
        s = lax.axis_index("s")
        gsl = pl.ds(c * d, d)
        cn16 = jnp.full((LANES,), c * n, jnp.int32)

        _zero_vmem_2d(zed_v, ZR, d)

        @pl.when(s < NS - 1)
        def _():
            for off, sz in _chunk_list(ch, ZR):
                pltpu.sync_copy(zed_v.at[pl.ds(0, sz)],
                                acc_sh.at[pl.ds(s * ch + off, sz)])

        @pl.when(s == NS - 1)
        def _():
            for off, sz in _chunk_list(last, ZR):
                pltpu.sync_copy(zed_v.at[pl.ds(0, sz)],
                                acc_sh.at[pl.ds((NS - 1) * ch + off, sz)])

        def offset_rows(buf):
            def obody(q, carry):
                r = q // (K_BLK // LANES)
                o = (q % (K_BLK // LANES)) * LANES
                sl = pl.ds(o, LANES)
                row_v[buf, r, sl] = row_v[buf, r, sl] + cn16
                return carry
            lax.fori_loop(0, GI * K_BLK // LANES, obody, 0)

        def stage_group(g, buf, sem):
            pltpu.async_copy(row_hbm.at[s, pl.ds(g * GI, GI)], row_v.at[buf], sem)
            pltpu.async_copy(col_hbm.at[s, pl.ds(g * GI, GI)], col_v.at[buf], sem)
            pltpu.async_copy(w_hbm.at[s, pl.ds(g * GI, GI)], w_v.at[buf], sem)

        def wait_group(g, buf, sem):
            pltpu.make_async_copy(row_hbm.at[s, pl.ds(g * GI, GI)],
                                  row_v.at[buf], sem).wait()
            pltpu.make_async_copy(col_hbm.at[s, pl.ds(g * GI, GI)],
                                  col_v.at[buf], sem).wait()
            pltpu.make_async_copy(w_hbm.at[s, pl.ds(g * GI, GI)],
                                  w_v.at[buf], sem).wait()

        stage_group(0, 0, isem[0])
        wait_group(0, 0, isem[0])
        offset_rows(0)
        plsc.subcore_barrier()

        for b in range(2):
            pltpu.async_copy(g_hbm.at[row_v.at[0, b]], rows_v.at[b], gsem[b])

        def grouppair(g0, carry):
            for bgi in range(2):
                g = g0 * 2 + bgi
                for jb in range(GI):
                    b = jb % 2
                    j = g * GI + jb
                    pltpu.make_async_copy(g_hbm.at[row_v.at[bgi, jb]],
                                          rows_v.at[b], gsem[b]).wait()
                    if jb >= 2:
                        cprev = col_v.at[bgi, jb - 2]
                    else:
                        cprev = col_v.at[1 - bgi, GI - 2 + jb]

                    @pl.when(j >= 2)
                    def _(cprev=cprev, b=b):
                        pltpu.make_async_copy(sc_v.at[b], acc_sh.at[cprev],
                                              ssem[b]).wait()

                    def scale(kb, carry2, bgi=bgi, jb=jb, b=b):
                        w16 = w_v[bgi, jb, pl.ds(kb * LANES, LANES)]
                        for jj in range(LANES):
                            wk = _lane_bcast(w16, jj)
                            _scale_store(rows_v, sc_v, b, kb * LANES + jj, d,
                                         wk)
                        return carry2

                    lax.fori_loop(0, K_BLK // LANES, scale, 0)
                    pltpu.async_copy(sc_v.at[b], acc_sh.at[col_v.at[bgi, jb]],
                                     ssem[b], add=True)

                    if jb == 2:
                        @pl.when(g + 1 < ngrp)
                        def _(g=g, bgi=bgi):
                            stage_group(g + 1, 1 - bgi, isem[1 - bgi])

                    if jb == GI - 2:
                        @pl.when(g + 1 < ngrp)
                        def _(g=g, bgi=bgi):
                            wait_group(g + 1, 1 - bgi, isem[1 - bgi])
                            offset_rows(1 - bgi)

                    if jb <= GI - 3:
                        nidx = row_v.at[bgi, jb + 2]
                    else:
                        nidx = row_v.at[1 - bgi, jb + 2 - GI]

                    @pl.when(j + 2 < nblk2)
                    def _(nidx=nidx, b=b):
                        pltpu.async_copy(g_hbm.at[nidx], rows_v.at[b], gsem[b])
            return carry

        lax.fori_loop(0, ngrp // 2, grouppair, 0)
        for b in range(2):
            pltpu.make_async_copy(sc_v.at[b],
                                  acc_sh.at[col_v.at[1, GI - 2 + b]],
                                  ssem[b]).wait()
        plsc.subcore_barrier()

        @pl.when(s < NS - 1)
        def _():
            for off, sz in _chunk_list(ch, ZR):
                pltpu.sync_copy(acc_sh.at[pl.ds(s * ch + off, sz)],
                                zed_v.at[pl.ds(0, sz)])
                pltpu.sync_copy(zed_v.at[pl.ds(0, sz)],
                                out_hbm.at[pl.ds(s * ch + off, sz), gsl])

        @pl.when(s == NS - 1)
        def _():
            for off, sz in _chunk_list(last, ZR):
                pltpu.sync_copy(acc_sh.at[pl.ds((NS - 1) * ch + off, sz)],
                                zed_v.at[pl.ds(0, sz)])
                pltpu.sync_copy(zed_v.at[pl.ds(0, sz)],
                                out_hbm.at[pl.ds((NS - 1) * ch + off, sz), gsl])

    return edge1_kernel


def _dis_from(deg_ref):
    deg = deg_ref[:, 0] + deg_ref[:, 1] + 1.0
    return jnp.where(deg > 0, lax.rsqrt(deg), 0.0)


def _make_prep_body(dch):
    def body(deg_ref, x_ref, w_ref, gf_ref, gcm_ref):
        dis = _dis_from(deg_ref)
        h = jnp.dot(x_ref[...], w_ref[...], preferred_element_type=jnp.float32)
        g = h * dis[:, None]
        gf_ref[...] = g
        for p in range(NC):
            gcm_ref[p] = _pack_bf16_pairs(g[:, p * dch:(p + 1) * dch])
    return body


def _mid_body(deg_ref, s_ref, g1_ref, b1_ref, w2_ref, g2_ref, g2b_ref):
    dis = _dis_from(deg_ref)
    t = (s_ref[...] + g1_ref[...]) * dis[:, None] + b1_ref[...]
    t = jnp.maximum(t, 0.0)
    h2 = jnp.dot(t, w2_ref[...], preferred_element_type=jnp.float32)
    g2 = h2 * dis[:, None]
    g2_ref[...] = g2
    g2b_ref[...] = _pack_bf16_pairs(g2)


def _make_fin_body(d2):
    def body(deg_ref, s_ref, g2_ref, b2_ref, o_ref):
        dis = _dis_from(deg_ref)
        s = s_ref[:, :d2] + s_ref[:, d2:NC * d2]
        o_ref[...] = (s + g2_ref[...]) * dis[:, None] + b2_ref[...]
    return body


def _row_blocks(n):
    for blk in (2000, 1000, 500, 250, 125, n):
        if n % blk == 0 and blk % 8 == 0:
            return blk, n // blk
    return n, 1


def kernel(x, edge_index, edge_weight, W1, b1, W2, b2):
    n, f = x.shape
    d1 = W1.shape[1]
    d2 = W2.shape[1]
    e = edge_weight.shape[0]

    row = edge_index[0].astype(jnp.int32)
    col = edge_index[1].astype(jnp.int32)
    w = edge_weight.astype(jnp.float32)

    emult = NS * K_BLK * GI * 2
    e_pad = (e + emult - 1) // emult * emult
    if e_pad != e:
        extra = e_pad - e
        pad_idx = (jnp.arange(extra, dtype=jnp.int32) * 16) % n
        row = jnp.concatenate([row, pad_idx])
        col = jnp.concatenate([col, pad_idx])
        w = jnp.concatenate([w, jnp.zeros((extra,), jnp.float32)])
    nblk = e_pad // (NW * K_BLK)
    nblk2 = e_pad // (NS * K_BLK)
    row3 = row.reshape(NW, nblk, K_BLK)
    col3 = col.reshape(NW, nblk, K_BLK)
    w3 = w.reshape(NW, nblk, K_BLK)
    row2 = row.reshape(NS, nblk2, K_BLK)
    col2 = col.reshape(NS, nblk2, K_BLK)
    w2 = w.reshape(NS, nblk2, K_BLK)

    deg_parts = _make_deg_kernel(n, nblk)(col3, w3)
    deg_nt = deg_parts.reshape(NC, n).T

    blk, nrblk = _row_blocks(n)
    full2 = lambda i: (0, 0)

    DCH = d1 // NC
    assert d1 == NC * DCH and d2 == DCH

    g1f, g1cm = pl.pallas_call(
        _make_prep_body(DCH),
        grid=(nrblk,),
        in_specs=[
            pl.BlockSpec((blk, NC), lambda i: (i, 0)),
            pl.BlockSpec((blk, f), lambda i: (i, 0)),
            pl.BlockSpec((f, d1), full2),
        ],
        out_specs=[
            pl.BlockSpec((blk, d1), lambda i: (i, 0)),
            pl.BlockSpec((NC, blk, DCH // 2), lambda i: (0, i, 0)),
        ],
        out_shape=[
            jax.ShapeDtypeStruct((n, d1), jnp.float32),
            jax.ShapeDtypeStruct((NC, n, DCH // 2), jnp.int32),
        ],
    )(deg_nt, x, W1)

    s1 = _make_edge1_kernel(n, nblk2, DCH)(
        g1cm.reshape(NC * n, DCH // 2), row2, col2, w2)

    g2, g2b = pl.pallas_call(
        _mid_body,
        grid=(nrblk,),
        in_specs=[
            pl.BlockSpec((blk, NC), lambda i: (i, 0)),
            pl.BlockSpec((blk, d1), lambda i: (i, 0)),
            pl.BlockSpec((blk, d1), lambda i: (i, 0)),
            pl.BlockSpec((1, d1), full2),
            pl.BlockSpec((d1, d2), full2),
        ],
        out_specs=[
            pl.BlockSpec((blk, d2), lambda i: (i, 0)),
            pl.BlockSpec((blk, d2 // 2), lambda i: (i, 0)),
        ],
        out_shape=[
            jax.ShapeDtypeStruct((n, d2), jnp.float32),
            jax.ShapeDtypeStruct((n, d2 // 2), jnp.int32),
        ],
    )(deg_nt, s1, g1f, b1.reshape(1, d1), W2)

    s2 = _make_edge_kernel(n, nblk, d2)(g2b, row3, col3, w3)

    out = pl.pallas_call(
        _make_fin_body(d2),
        grid=(nrblk,),
        in_specs=[
            pl.BlockSpec((blk, NC), lambda i: (i, 0)),
            pl.BlockSpec((blk, NC * d2), lambda i: (i, 0)),
            pl.BlockSpec((blk, d2), lambda i: (i, 0)),
            pl.BlockSpec((1, d2), full2),
        ],
        out_specs=pl.BlockSpec((blk, d2), lambda i: (i, 0)),
        out_shape=jax.ShapeDtypeStruct((n, d2), jnp.float32),
    )(deg_nt, s2, g2, b2.reshape(1, d2))

    return out


# speedup vs baseline: 1.9982x; 1.7826x over previous
import functools

import jax
import jax.numpy as jnp
from jax import lax
from jax.experimental import pallas as pl
from jax.experimental.pallas import tpu as pltpu
from jax.experimental.pallas import tpu_sc as plsc

NC = 2
NS = 16
NW = NC * NS
LANES = 16
K_BLK = 128


def _tile_slices(n):
    ch = (((n + NS - 1) // NS) + 7) // 8 * 8
    last = n - (NS - 1) * ch
    assert 0 < last <= ch and ch % 8 == 0 and last % 8 == 0
    return ch, last


def _lane_bcast(vec, lane):
    idx = jnp.full((LANES, 1), lane, jnp.int32)
    dnums = lax.GatherDimensionNumbers(
        offset_dims=(), collapsed_slice_dims=(0,), start_index_map=(0,))
    return lax.gather(vec, idx, dnums, (1,),
                      mode=lax.GatherScatterMode.PROMISE_IN_BOUNDS)


def _zero_vmem_2d(ref, rows, d):
    zero16 = jnp.zeros((LANES,), jnp.float32)

    def body(r, carry):
        for q in range(d // LANES):
            ref[r, pl.ds(q * LANES, LANES)] = zero16
        return carry

    lax.fori_loop(0, rows, body, 0)


def _zero_vmem_1d(ref, total):
    zero16 = jnp.zeros((LANES,), jnp.float32)

    def body(i, carry):
        ref[pl.ds(i * LANES, LANES)] = zero16
        return carry

    lax.fori_loop(0, total // LANES, body, 0)


def _make_deg_kernel(n, nblk):
    ch, last = _tile_slices(n)
    chz = (ch + LANES - 1) // LANES * LANES
    mesh = plsc.VectorSubcoreMesh(core_axis_name="c", subcore_axis_name="s")

    @functools.partial(
        pl.kernel,
        out_type=jax.ShapeDtypeStruct((NC * n,), jnp.float32),
        mesh=mesh,
        scratch_types=[
            pltpu.VMEM((nblk, K_BLK), jnp.int32),
            pltpu.VMEM((nblk, K_BLK), jnp.float32),
            pltpu.VMEM((chz,), jnp.float32),
            pltpu.VMEM_SHARED((n,), jnp.float32),
            pltpu.SemaphoreType.DMA,
        ],
        compiler_params=pltpu.CompilerParams(use_tc_tiling_on_sc=False),
    )
    def deg_kernel(col_hbm, w_hbm, out_hbm, col_v, w_v, zed_v, acc_sh, sem):
        c = lax.axis_index("c")
        s = lax.axis_index("s")
        wid = c * NS + s

        _zero_vmem_1d(zed_v, chz)

        @pl.when(s < NS - 1)
        def _():
            pltpu.sync_copy(zed_v.at[pl.ds(0, ch)], acc_sh.at[pl.ds(s * ch, ch)])

        @pl.when(s == NS - 1)
        def _():
            pltpu.sync_copy(zed_v.at[pl.ds(0, last)],
                            acc_sh.at[pl.ds((NS - 1) * ch, last)])

        pltpu.sync_copy(col_hbm.at[wid], col_v)
        pltpu.sync_copy(w_hbm.at[wid], w_v)
        plsc.subcore_barrier()

        GRP = 8

        def grp(gg, carry):
            for b in range(GRP):
                pltpu.async_copy(w_v.at[gg * GRP + b],
                                 acc_sh.at[col_v.at[gg * GRP + b]], sem,
                                 add=True)
            for b in range(GRP):
                pltpu.make_async_copy(w_v.at[gg * GRP + b],
                                      acc_sh.at[col_v.at[gg * GRP + b]],
                                      sem).wait()
            return carry

        assert nblk % GRP == 0
        lax.fori_loop(0, nblk // GRP, grp, 0)
        plsc.subcore_barrier()

        @pl.when(s < NS - 1)
        def _():
            pltpu.sync_copy(acc_sh.at[pl.ds(s * ch, ch)], zed_v.at[pl.ds(0, ch)])
            pltpu.sync_copy(zed_v.at[pl.ds(0, ch)],
                            out_hbm.at[pl.ds(c * n + s * ch, ch)])

        @pl.when(s == NS - 1)
        def _():
            pltpu.sync_copy(acc_sh.at[pl.ds((NS - 1) * ch, last)],
                            zed_v.at[pl.ds(0, last)])
            pltpu.sync_copy(zed_v.at[pl.ds(0, last)],
                            out_hbm.at[pl.ds(c * n + (NS - 1) * ch, last)])

    return deg_kernel


def _chunk_list(total, zr):
    k, rem = divmod(total, zr)
    return [(i * zr, zr) for i in range(k)] + ([(k * zr, rem)] if rem else [])


ZR = 128


def _make_edge_kernel(n, nblk, d):
    assert d % LANES == 0 and nblk % 2 == 0
    ch, last = _tile_slices(n)
    mesh = plsc.VectorSubcoreMesh(core_axis_name="c", subcore_axis_name="s")

    @functools.partial(
        pl.kernel,
        out_type=jax.ShapeDtypeStruct((n, NC * d), jnp.float32),
        mesh=mesh,
        scratch_types=[
            pltpu.VMEM((nblk, K_BLK), jnp.int32),
            pltpu.VMEM((nblk, K_BLK), jnp.int32),
            pltpu.VMEM((nblk, K_BLK), jnp.float32),
            pltpu.VMEM((2, K_BLK, d), jnp.float32),
            pltpu.VMEM((2, K_BLK, d), jnp.float32),
            pltpu.VMEM((ZR, d), jnp.float32),
            pltpu.VMEM_SHARED((n, d), jnp.float32),
            pltpu.SemaphoreType.DMA,
            pltpu.SemaphoreType.DMA,
            pltpu.SemaphoreType.DMA,
            pltpu.SemaphoreType.DMA,
        ],
        compiler_params=pltpu.CompilerParams(use_tc_tiling_on_sc=False),
    )
    def edge_kernel(g_hbm, row_hbm, col_hbm, w_hbm, out_hbm,
                    row_v, col_v, w_v, rows_v, sc_v, zed_v, acc_sh,
                    gsem0, gsem1, ssem0, ssem1):
        gsem = (gsem0, gsem1)
        ssem = (ssem0, ssem1)
        c = lax.axis_index("c")
        s = lax.axis_index("s")
        wid = c * NS + s

        _zero_vmem_2d(zed_v, ZR, d)

        @pl.when(s < NS - 1)
        def _():
            for off, sz in _chunk_list(ch, ZR):
                pltpu.sync_copy(zed_v.at[pl.ds(0, sz)],
                                acc_sh.at[pl.ds(s * ch + off, sz)])

        @pl.when(s == NS - 1)
        def _():
            for off, sz in _chunk_list(last, ZR):
                pltpu.sync_copy(zed_v.at[pl.ds(0, sz)],
                                acc_sh.at[pl.ds((NS - 1) * ch + off, sz)])

        pltpu.sync_copy(row_hbm.at[wid], row_v)
        pltpu.sync_copy(col_hbm.at[wid], col_v)
        pltpu.sync_copy(w_hbm.at[wid], w_v)
        plsc.subcore_barrier()

        for b in range(2):
            pltpu.async_copy(g_hbm.at[row_v.at[b]], rows_v.at[b], gsem[b])

        def blk2(j0, carry):
            for b in range(2):
                j = j0 * 2 + b
                pltpu.make_async_copy(g_hbm.at[row_v.at[j]], rows_v.at[b],
                                      gsem[b]).wait()

                @pl.when(j0 > 0)
                def _():
                    jp = j - 2
                    pltpu.make_async_copy(sc_v.at[b],
                                          acc_sh.at[col_v.at[jp]],
                                          ssem[b]).wait()

                def scale(kb, carry2):
                    w16 = w_v[j, pl.ds(kb * LANES, LANES)]
                    for jj in range(LANES):
                        wk = _lane_bcast(w16, jj)
                        k = kb * LANES + jj
                        for dd in range(d // LANES):
                            sl = pl.ds(dd * LANES, LANES)
                            sc_v[b, k, sl] = rows_v[b, k, sl] * wk
                    return carry2

                lax.fori_loop(0, K_BLK // LANES, scale, 0)
                pltpu.async_copy(sc_v.at[b], acc_sh.at[col_v.at[j]],
                                 ssem[b], add=True)

                @pl.when(j + 2 < nblk)
                def _():
                    pltpu.async_copy(g_hbm.at[row_v.at[j + 2]], rows_v.at[b],
                                     gsem[b])
            return carry

        lax.fori_loop(0, nblk // 2, blk2, 0)
        for b in range(2):
            pltpu.make_async_copy(sc_v.at[b],
                                  acc_sh.at[col_v.at[nblk - 2 + b]],
                                  ssem[b]).wait()
        plsc.subcore_barrier()

        osl = pl.ds(c * d, d)

        @pl.when(s < NS - 1)
        def _():
            for off, sz in _chunk_list(ch, ZR):
                pltpu.sync_copy(acc_sh.at[pl.ds(s * ch + off, sz)],
                                zed_v.at[pl.ds(0, sz)])
                pltpu.sync_copy(zed_v.at[pl.ds(0, sz)],
                                out_hbm.at[pl.ds(s * ch + off, sz), osl])

        @pl.when(s == NS - 1)
        def _():
            for off, sz in _chunk_list(last, ZR):
                pltpu.sync_copy(acc_sh.at[pl.ds((NS - 1) * ch + off, sz)],
                                zed_v.at[pl.ds(0, sz)])
                pltpu.sync_copy(zed_v.at[pl.ds(0, sz)],
                                out_hbm.at[pl.ds((NS - 1) * ch + off, sz), osl])

    return edge_kernel


GI = 8


def _make_edge1_kernel(n, nblk2, d):
    assert d % LANES == 0 and nblk2 % (2 * GI) == 0 and nblk2 // GI >= 2
    ch, last = _tile_slices(n)
    ngrp = nblk2 // GI
    mesh = plsc.VectorSubcoreMesh(core_axis_name="c", subcore_axis_name="s")

    @functools.partial(
        pl.kernel,
        out_type=jax.ShapeDtypeStruct((n, NC * d), jnp.float32),
        mesh=mesh,
        scratch_types=[
            pltpu.VMEM((2, GI, K_BLK), jnp.int32),
            pltpu.VMEM((2, GI, K_BLK), jnp.int32),
            pltpu.VMEM((2, GI, K_BLK), jnp.float32),
            pltpu.VMEM((2, K_BLK, d), jnp.float32),
            pltpu.VMEM((2, K_BLK, d), jnp.float32),
            pltpu.VMEM((ZR, d), jnp.float32),
            pltpu.VMEM_SHARED((n, d), jnp.float32),
            pltpu.SemaphoreType.DMA,
            pltpu.SemaphoreType.DMA,
            pltpu.SemaphoreType.DMA,
            pltpu.SemaphoreType.DMA,
            pltpu.SemaphoreType.DMA,
            pltpu.SemaphoreType.DMA,
        ],
        compiler_params=pltpu.CompilerParams(use_tc_tiling_on_sc=False),
    )
    def edge1_kernel(g_hbm, row_hbm, col_hbm, w_hbm, out_hbm,
                     row_v, col_v, w_v, rows_v, sc_v, zed_v, acc_sh,
                     gsem0, gsem1, ssem0, ssem1, isem0, isem1):
        gsem = (gsem0, gsem1)
        ssem = (ssem0, ssem1)
        isem = (isem0, isem1)
        c = lax.axis_index("c")
        s = lax.axis_index("s")
        gsl = pl.ds(c * d, d)
        cn16 = jnp.full((LANES,), c * n, jnp.int32)

        _zero_vmem_2d(zed_v, ZR, d)

        @pl.when(s < NS - 1)
        def _():
            for off, sz in _chunk_list(ch, ZR):
                pltpu.sync_copy(zed_v.at[pl.ds(0, sz)],
                                acc_sh.at[pl.ds(s * ch + off, sz)])

        @pl.when(s == NS - 1)
        def _():
            for off, sz in _chunk_list(last, ZR):
                pltpu.sync_copy(zed_v.at[pl.ds(0, sz)],
                                acc_sh.at[pl.ds((NS - 1) * ch + off, sz)])

        def offset_rows(buf):
            def obody(q, carry):
                r = q // (K_BLK // LANES)
                o = (q % (K_BLK // LANES)) * LANES
                sl = pl.ds(o, LANES)
                row_v[buf, r, sl] = row_v[buf, r, sl] + cn16
                return carry
            lax.fori_loop(0, GI * K_BLK // LANES, obody, 0)

        def stage_group(g, buf, sem):
            pltpu.async_copy(row_hbm.at[s, pl.ds(g * GI, GI)], row_v.at[buf], sem)
            pltpu.async_copy(col_hbm.at[s, pl.ds(g * GI, GI)], col_v.at[buf], sem)
            pltpu.async_copy(w_hbm.at[s, pl.ds(g * GI, GI)], w_v.at[buf], sem)

        def wait_group(g, buf, sem):
            pltpu.make_async_copy(row_hbm.at[s, pl.ds(g * GI, GI)],
                                  row_v.at[buf], sem).wait()
            pltpu.make_async_copy(col_hbm.at[s, pl.ds(g * GI, GI)],
                                  col_v.at[buf], sem).wait()
            pltpu.make_async_copy(w_hbm.at[s, pl.ds(g * GI, GI)],
                                  w_v.at[buf], sem).wait()

        stage_group(0, 0, isem[0])
        wait_group(0, 0, isem[0])
        offset_rows(0)
        plsc.subcore_barrier()

        for b in range(2):
            pltpu.async_copy(g_hbm.at[row_v.at[0, b]], rows_v.at[b], gsem[b])

        def grouppair(g0, carry):
            for bgi in range(2):
                g = g0 * 2 + bgi
                for jb in range(GI):
                    b = jb % 2
                    j = g * GI + jb
                    pltpu.make_async_copy(g_hbm.at[row_v.at[bgi, jb]],
                                          rows_v.at[b], gsem[b]).wait()
                    if jb >= 2:
                        cprev = col_v.at[bgi, jb - 2]
                    else:
                        cprev = col_v.at[1 - bgi, GI - 2 + jb]

                    @pl.when(j >= 2)
                    def _(cprev=cprev, b=b):
                        pltpu.make_async_copy(sc_v.at[b], acc_sh.at[cprev],
                                              ssem[b]).wait()

                    def scale(kb, carry2, bgi=bgi, jb=jb, b=b):
                        w16 = w_v[bgi, jb, pl.ds(kb * LANES, LANES)]
                        for jj in range(LANES):
                            wk = _lane_bcast(w16, jj)
                            k = kb * LANES + jj
                            for dd in range(d // LANES):
                                sl = pl.ds(dd * LANES, LANES)
                                sc_v[b, k, sl] = rows_v[b, k, sl] * wk
                        return carry2

                    lax.fori_loop(0, K_BLK // LANES, scale, 0)
                    pltpu.async_copy(sc_v.at[b], acc_sh.at[col_v.at[bgi, jb]],
                                     ssem[b], add=True)

                    if jb == 2:
                        @pl.when(g + 1 < ngrp)
                        def _(g=g, bgi=bgi):
                            stage_group(g + 1, 1 - bgi, isem[1 - bgi])

                    if jb == GI - 2:
                        @pl.when(g + 1 < ngrp)
                        def _(g=g, bgi=bgi):
                            wait_group(g + 1, 1 - bgi, isem[1 - bgi])
                            offset_rows(1 - bgi)

                    if jb <= GI - 3:
                        nidx = row_v.at[bgi, jb + 2]
                    else:
                        nidx = row_v.at[1 - bgi, jb + 2 - GI]

                    @pl.when(j + 2 < nblk2)
                    def _(nidx=nidx, b=b):
                        pltpu.async_copy(g_hbm.at[nidx], rows_v.at[b], gsem[b])
            return carry

        lax.fori_loop(0, ngrp // 2, grouppair, 0)
        for b in range(2):
            pltpu.make_async_copy(sc_v.at[b],
                                  acc_sh.at[col_v.at[1, GI - 2 + b]],
                                  ssem[b]).wait()
        plsc.subcore_barrier()

        @pl.when(s < NS - 1)
        def _():
            for off, sz in _chunk_list(ch, ZR):
                pltpu.sync_copy(acc_sh.at[pl.ds(s * ch + off, sz)],
                                zed_v.at[pl.ds(0, sz)])
                pltpu.sync_copy(zed_v.at[pl.ds(0, sz)],
                                out_hbm.at[pl.ds(s * ch + off, sz), gsl])

        @pl.when(s == NS - 1)
        def _():
            for off, sz in _chunk_list(last, ZR):
                pltpu.sync_copy(acc_sh.at[pl.ds((NS - 1) * ch + off, sz)],
                                zed_v.at[pl.ds(0, sz)])
                pltpu.sync_copy(zed_v.at[pl.ds(0, sz)],
                                out_hbm.at[pl.ds((NS - 1) * ch + off, sz), gsl])

    return edge1_kernel


def _make_marshal_body(n, e, e_pad, nblk2):
    extra = e_pad - e

    def body(ei_ref, w_ref, row_ref, col_ref, w_out_ref):
        ei = ei_ref[...]
        wv = w_ref[...]
        if extra:
            pad2 = (lax.broadcasted_iota(jnp.int32, (2, extra), 1) * 16) % n
            ei = jnp.concatenate([ei, pad2], axis=1)
            wv = jnp.concatenate([wv, jnp.zeros((1, extra), jnp.float32)],
                                 axis=1)
        row_ref[...] = ei[0].reshape(NS, nblk2, K_BLK)
        col_ref[...] = ei[1].reshape(NS, nblk2, K_BLK)
        w_out_ref[...] = wv[0].reshape(NS, nblk2, K_BLK)

    return body


def _dis_from(deg_ref):
    deg = deg_ref[:, 0] + deg_ref[:, 1] + 1.0
    return jnp.where(deg > 0, lax.rsqrt(deg), 0.0)


def _make_prep_body(dch):
    def body(deg_ref, x_ref, w_ref, gf_ref, gcm_ref):
        dis = _dis_from(deg_ref)
        h = jnp.dot(x_ref[...], w_ref[...], preferred_element_type=jnp.float32)
        g = h * dis[:, None]
        gf_ref[...] = g
        for p in range(NC):
            gcm_ref[p] = g[:, p * dch:(p + 1) * dch]
    return body


def _mid_body(deg_ref, s_ref, g1_ref, b1_ref, w2_ref, g2_ref):
    dis = _dis_from(deg_ref)
    t = (s_ref[...] + g1_ref[...]) * dis[:, None] + b1_ref[...]
    t = jnp.maximum(t, 0.0)
    h2 = jnp.dot(t, w2_ref[...], preferred_element_type=jnp.float32)
    g2_ref[...] = h2 * dis[:, None]


def _make_fin_body(d2):
    def body(deg_ref, s_ref, g2_ref, b2_ref, o_ref):
        dis = _dis_from(deg_ref)
        s = s_ref[:, :d2] + s_ref[:, d2:NC * d2]
        o_ref[...] = (s + g2_ref[...]) * dis[:, None] + b2_ref[...]
    return body


def _row_blocks(n):
    for blk in (2000, 1000, 500, 250, 125, n):
        if n % blk == 0 and blk % 8 == 0:
            return blk, n // blk
    return n, 1


def kernel(x, edge_index, edge_weight, W1, b1, W2, b2):
    n, f = x.shape
    d1 = W1.shape[1]
    d2 = W2.shape[1]
    e = edge_weight.shape[0]

    emult = NS * K_BLK * GI * 2
    e_pad = (e + emult - 1) // emult * emult
    nblk = e_pad // (NW * K_BLK)
    nblk2 = e_pad // (NS * K_BLK)

    idx3 = pl.BlockSpec((NS, nblk2, K_BLK), lambda: (0, 0, 0))
    row2, col2, w2 = pl.pallas_call(
        _make_marshal_body(n, e, e_pad, nblk2),
        in_specs=[
            pl.BlockSpec((NC, e), lambda: (0, 0)),
            pl.BlockSpec((1, e), lambda: (0, 0)),
        ],
        out_specs=[idx3, idx3, idx3],
        out_shape=[
            jax.ShapeDtypeStruct((NS, nblk2, K_BLK), jnp.int32),
            jax.ShapeDtypeStruct((NS, nblk2, K_BLK), jnp.int32),
            jax.ShapeDtypeStruct((NS, nblk2, K_BLK), jnp.float32),
        ],
    )(edge_index.astype(jnp.int32), edge_weight.astype(jnp.float32).reshape(1, e))
    row3 = row2.reshape(NW, nblk, K_BLK)
    col3 = col2.reshape(NW, nblk, K_BLK)
    w3 = w2.reshape(NW, nblk, K_BLK)

    deg_parts = _make_deg_kernel(n, nblk)(col3, w3)
    deg_nt = deg_parts.reshape(NC, n).T

    blk, nrblk = _row_blocks(n)
    full2 = lambda i: (0, 0)

    DCH = d1 // NC
    assert d1 == NC * DCH and d2 == DCH

    g1f, g1cm = pl.pallas_call(
        _make_prep_body(DCH),
        grid=(nrblk,),
        in_specs=[
            pl.BlockSpec((blk, NC), lambda i: (i, 0)),
            pl.BlockSpec((blk, f), lambda i: (i, 0)),
            pl.BlockSpec((f, d1), full2),
        ],
        out_specs=[
            pl.BlockSpec((blk, d1), lambda i: (i, 0)),
            pl.BlockSpec((NC, blk, DCH), lambda i: (0, i, 0)),
        ],
        out_shape=[
            jax.ShapeDtypeStruct((n, d1), jnp.float32),
            jax.ShapeDtypeStruct((NC, n, DCH), jnp.float32),
        ],
    )(deg_nt, x, W1)

    s1 = _make_edge1_kernel(n, nblk2, DCH)(
        g1cm.reshape(NC * n, DCH), row2, col2, w2)

    g2 = pl.pallas_call(
        _mid_body,
        grid=(nrblk,),
        in_specs=[
            pl.BlockSpec((blk, NC), lambda i: (i, 0)),
            pl.BlockSpec((blk, d1), lambda i: (i, 0)),
            pl.BlockSpec((blk, d1), lambda i: (i, 0)),
            pl.BlockSpec((1, d1), full2),
            pl.BlockSpec((d1, d2), full2),
        ],
        out_specs=pl.BlockSpec((blk, d2), lambda i: (i, 0)),
        out_shape=jax.ShapeDtypeStruct((n, d2), jnp.float32),
    )(deg_nt, s1, g1f, b1.reshape(1, d1), W2)

    s2 = _make_edge_kernel(n, nblk, d2)(g2, row3, col3, w3)

    out = pl.pallas_call(
        _make_fin_body(d2),
        grid=(nrblk,),
        in_specs=[
            pl.BlockSpec((blk, NC), lambda i: (i, 0)),
            pl.BlockSpec((blk, NC * d2), lambda i: (i, 0)),
            pl.BlockSpec((blk, d2), lambda i: (i, 0)),
            pl.BlockSpec((1, d2), full2),
        ],
        out_specs=pl.BlockSpec((blk, d2), lambda i: (i, 0)),
        out_shape=jax.ShapeDtypeStruct((n, d2), jnp.float32),
    )(deg_nt, s2, g2, b2.reshape(1, d2))

    return out
